# baseline (speedup 1.0000x reference)
import functools

import jax
import jax.numpy as jnp
from jax.experimental import pallas as pl
from jax.experimental.pallas import tpu as pltpu

_BR = 256
_CS = 128
_RS = 64
_M = 3

_BIG = 3e38


def _sq_kernel(x_ref, sq_ref):
    xb = x_ref[:]
    sq_ref[:] = jnp.sum(xb * xb, axis=1, keepdims=True)


def _knn_conv_kernel(xb_ref, x_ref, sqb_ref, sqr_ref, w_ref, b_ref,
                     o_ref, msg_vmem, idx_vmem, idx_smem, sem, *, k, br, n):
    p = pl.program_id(0)
    g = jax.lax.dot_general(
        xb_ref[:], x_ref[:], (((1,), (1,)), ((), ())),
        preferred_element_type=jnp.float32)
    scores = (sqb_ref[:] + sqr_ref[:]) - 2.0 * g
    col = jax.lax.broadcasted_iota(jnp.int32, (br, n), 1)
    row = jax.lax.broadcasted_iota(jnp.int32, (br, n), 0)
    scores = scores + jnp.where(col == row + p * br, 1e10, 0.0)

    nc = n // _CS
    lane = jax.lax.broadcasted_iota(jnp.int32, (_RS, _CS), 1)
    v_rows = []
    gi_rows = []
    for r0 in range(0, br, _RS):
        t0 = jnp.full((_RS, _CS), _BIG)
        t1 = jnp.full((_RS, _CS), _BIG)
        t2 = jnp.full((_RS, _CS), _BIG)
        i0 = jnp.zeros((_RS, _CS), jnp.int32)
        i1 = jnp.zeros((_RS, _CS), jnp.int32)
        i2 = jnp.zeros((_RS, _CS), jnp.int32)
        for c in range(nc):
            v = jax.lax.slice(scores, (r0, c * _CS), (r0 + _RS, (c + 1) * _CS))
            cc = jnp.full((_RS, _CS), c, jnp.int32)
            lt2 = v < t2
            cv = jnp.where(lt2, v, t2)
            ci = jnp.where(lt2, cc, i2)
            lt1 = cv < t1
            lt0 = cv < t0
            t2 = jnp.where(lt1, t1, cv)
            i2 = jnp.where(lt1, i1, ci)
            t1 = jnp.where(lt0, t0, jnp.where(lt1, cv, t1))
            i1 = jnp.where(lt0, i0, jnp.where(lt1, ci, i1))
            t0 = jnp.where(lt0, cv, t0)
            i0 = jnp.where(lt0, ci, i0)
        v_rows.append(jnp.concatenate([t0, t1, t2], axis=1))
        gi_rows.append(jnp.concatenate(
            [i0 * _CS + lane, i1 * _CS + lane, i2 * _CS + lane], axis=1))
    cand_v = jnp.concatenate(v_rows, axis=0)
    cand_g = jnp.concatenate(gi_rows, axis=0)

    mm = _M * _CS
    entry = jax.lax.broadcasted_iota(jnp.int32, (br, mm), 1)
    vk = None
    for t in range(k):
        ev = jnp.min(cand_v, axis=1, keepdims=True)
        gsel = jnp.min(jnp.where(cand_v == ev, cand_g, n), axis=1,
                       keepdims=True)
        idx_vmem[:, t:t + 1] = gsel
        if t == k - 1:
            vk = ev
        cand_v = jnp.where((cand_v == ev) & (cand_g == gsel), _BIG, cand_v)
    rem = jnp.min(cand_v, axis=1, keepdims=True)
    tie = (vk == rem).astype(jnp.int32)
    selm = (cand_v >= 1e30).astype(jnp.int32)
    lanefull = selm[:, 0:_CS]
    for j in range(1, _M):
        lanefull = lanefull * selm[:, j * _CS:(j + 1) * _CS]
    rowbad = jnp.maximum(jnp.max(lanefull, axis=1, keepdims=True), tie)
    allbad = jnp.max(rowbad, axis=0, keepdims=True)
    idx_vmem[:, k:k + 1] = jnp.broadcast_to(allbad, (br, 1))

    cp = pltpu.make_async_copy(idx_vmem, idx_smem, sem)
    cp.start()
    cp.wait()

    @pl.when(idx_smem[0, k] != 0)
    def _fallback():
        sc = scores
        for t in range(k):
            idxv = jnp.argmin(sc, axis=1, keepdims=True).astype(jnp.int32)
            idx_vmem[:, t:t + 1] = idxv
            if t < k - 1:
                sc = jnp.where(col == idxv, _BIG, sc)
        cp2 = pltpu.make_async_copy(idx_vmem, idx_smem, sem)
        cp2.start()
        cp2.wait()

    def body(i, _):
        xi = xb_ref[pl.ds(i, 1), :]
        for t in range(k):
            j = idx_smem[i, t]
            msg_vmem[pl.ds(t * br + i, 1), :] = x_ref[pl.ds(j, 1), :] - xi
        return 0

    jax.lax.fori_loop(0, br, body, 0)

    acc = None
    for t in range(k):
        ht = jax.lax.dot_general(
            msg_vmem[pl.ds(t * br, br), :], w_ref[:],
            (((1,), (1,)), ((), ())), preferred_element_type=jnp.float32)
        acc = ht if acc is None else jnp.maximum(acc, ht)
    o_ref[:] = acc + b_ref[:]


def _out_kernel(h_ref, w_ref, b_ref, o_ref):
    o_ref[:] = jax.lax.dot_general(
        h_ref[:], w_ref[:], (((1,), (1,)), ((), ())),
        preferred_element_type=jnp.float32) + b_ref[:]


def _edge_conv(x, W, b, k):
    n, d = x.shape
    h = W.shape[0]
    br = _BR
    sq = pl.pallas_call(
        _sq_kernel,
        grid=(n // br,),
        in_specs=[pl.BlockSpec((br, d), lambda p: (p, 0))],
        out_specs=pl.BlockSpec((br, 1), lambda p: (p, 0)),
        out_shape=jax.ShapeDtypeStruct((n, 1), jnp.float32),
    )(x)
    sqr = sq.reshape(1, n)
    b2d = b.reshape(1, h)
    out = pl.pallas_call(
        functools.partial(_knn_conv_kernel, k=k, br=br, n=n),
        grid=(n // br,),
        in_specs=[
            pl.BlockSpec((br, d), lambda p: (p, 0)),
            pl.BlockSpec((n, d), lambda p: (0, 0)),
            pl.BlockSpec((br, 1), lambda p: (p, 0)),
            pl.BlockSpec((1, n), lambda p: (0, 0)),
            pl.BlockSpec((h, d), lambda p: (0, 0)),
            pl.BlockSpec((1, h), lambda p: (0, 0)),
        ],
        out_specs=pl.BlockSpec((br, h), lambda p: (p, 0)),
        out_shape=jax.ShapeDtypeStruct((n, h), jnp.float32),
        scratch_shapes=[
            pltpu.VMEM((br * k, d), jnp.float32),
            pltpu.VMEM((br, k + 1), jnp.int32),
            pltpu.SMEM((br, k + 1), jnp.int32),
            pltpu.SemaphoreType.DMA,
        ],
    )(x, x, sq, sqr, W, b2d)
    return out


def kernel(x, W1, b1, W2, b2, W3, b3):
    h1 = _edge_conv(x, W1, b1, 5)
    h2 = _edge_conv(h1, W2, b2, 10)
    n, h = h2.shape
    c = W3.shape[0]
    br = _BR
    out = pl.pallas_call(
        _out_kernel,
        grid=(n // br,),
        in_specs=[
            pl.BlockSpec((br, h), lambda p: (p, 0)),
            pl.BlockSpec((c, h), lambda p: (0, 0)),
            pl.BlockSpec((1, c), lambda p: (0, 0)),
        ],
        out_specs=pl.BlockSpec((br, c), lambda p: (p, 0)),
        out_shape=jax.ShapeDtypeStruct((n, c), jnp.float32),
    )(h2, W3, b3.reshape(1, c))
    return out


# speedup vs baseline: 1.3725x; 1.3725x over previous
import functools

import jax
import jax.numpy as jnp
from jax.experimental import pallas as pl
from jax.experimental.pallas import tpu as pltpu

_BR = 256
_CS = 128
_RS = 64
_M = 4

_BIG = 3e38


def _sq_kernel(x_ref, sq_ref):
    xb = x_ref[:]
    sq_ref[:] = jnp.sum(xb * xb, axis=1, keepdims=True)


def _knn_conv_kernel(xb_ref, x_ref, sqb_ref, sqr_ref, w_ref, b_ref,
                     o_ref, msg_vmem, idx_vmem, idx_smem, sem, *, k, br, n):
    p = pl.program_id(0)
    g = jax.lax.dot_general(
        xb_ref[:], x_ref[:], (((1,), (1,)), ((), ())),
        preferred_element_type=jnp.float32)
    scores = (sqb_ref[:] + sqr_ref[:]) - 2.0 * g
    col = jax.lax.broadcasted_iota(jnp.int32, (br, n), 1)
    row = jax.lax.broadcasted_iota(jnp.int32, (br, n), 0)
    scores = scores + jnp.where(col == row + p * br, 1e10, 0.0)

    nc = n // _CS
    lane = jax.lax.broadcasted_iota(jnp.int32, (_RS, _CS), 1)
    v_rows = []
    gi_rows = []
    m = _M
    for r0 in range(0, br, _RS):
        T = [jnp.full((_RS, _CS), _BIG) for _ in range(m)]
        I = [jnp.zeros((_RS, _CS), jnp.int32) for _ in range(m)]
        for c in range(nc):
            v = jax.lax.slice(scores, (r0, c * _CS), (r0 + _RS, (c + 1) * _CS))
            cc = jnp.full((_RS, _CS), c, jnp.int32)
            ltb = v < T[m - 1]
            cv = jnp.where(ltb, v, T[m - 1])
            ci = jnp.where(ltb, cc, I[m - 1])
            lt = [cv < T[j] for j in range(m - 1)]
            T[m - 1] = jnp.where(lt[m - 2], T[m - 2], cv)
            I[m - 1] = jnp.where(lt[m - 2], I[m - 2], ci)
            for j in range(m - 2, 0, -1):
                T[j] = jnp.where(lt[j - 1], T[j - 1],
                                 jnp.where(lt[j], cv, T[j]))
                I[j] = jnp.where(lt[j - 1], I[j - 1],
                                 jnp.where(lt[j], ci, I[j]))
            T[0] = jnp.where(lt[0], cv, T[0])
            I[0] = jnp.where(lt[0], ci, I[0])
        v_rows.append(jnp.concatenate(T, axis=1))
        gi_rows.append(jnp.concatenate(
            [ij * _CS + lane for ij in I], axis=1))
    cand_v = jnp.concatenate(v_rows, axis=0)
    cand_g = jnp.concatenate(gi_rows, axis=0)

    mm = _M * _CS
    entry = jax.lax.broadcasted_iota(jnp.int32, (br, mm), 1)
    vk = None
    for t in range(k):
        ev = jnp.min(cand_v, axis=1, keepdims=True)
        gsel = jnp.min(jnp.where(cand_v == ev, cand_g, n), axis=1,
                       keepdims=True)
        idx_vmem[:, t:t + 1] = gsel
        if t == k - 1:
            vk = ev
        cand_v = jnp.where((cand_v == ev) & (cand_g == gsel), _BIG, cand_v)
    rem = jnp.min(cand_v, axis=1, keepdims=True)
    tie = (vk == rem).astype(jnp.int32)
    selm = (cand_v >= 1e30).astype(jnp.int32)
    lanefull = selm[:, 0:_CS]
    for j in range(1, _M):
        lanefull = lanefull * selm[:, j * _CS:(j + 1) * _CS]
    rowbad = jnp.maximum(jnp.max(lanefull, axis=1, keepdims=True), tie)
    allbad = jnp.max(rowbad, axis=0, keepdims=True)
    idx_vmem[:, k:k + 1] = jnp.broadcast_to(allbad, (br, 1))

    cp = pltpu.make_async_copy(idx_vmem, idx_smem, sem)
    cp.start()
    cp.wait()

    @pl.when(idx_smem[0, k] != 0)
    def _fallback():
        sc = scores
        for t in range(k):
            idxv = jnp.argmin(sc, axis=1, keepdims=True).astype(jnp.int32)
            idx_vmem[:, t:t + 1] = idxv
            if t < k - 1:
                sc = jnp.where(col == idxv, _BIG, sc)
        cp2 = pltpu.make_async_copy(idx_vmem, idx_smem, sem)
        cp2.start()
        cp2.wait()

    def body(i, _):
        xi = xb_ref[pl.ds(i, 1), :]
        for t in range(k):
            j = idx_smem[i, t]
            msg_vmem[pl.ds(t * br + i, 1), :] = x_ref[pl.ds(j, 1), :] - xi
        return 0

    jax.lax.fori_loop(0, br, body, 0)

    acc = None
    for t in range(k):
        ht = jax.lax.dot_general(
            msg_vmem[pl.ds(t * br, br), :], w_ref[:],
            (((1,), (1,)), ((), ())), preferred_element_type=jnp.float32)
        acc = ht if acc is None else jnp.maximum(acc, ht)
    o_ref[:] = acc + b_ref[:]


def _out_kernel(h_ref, w_ref, b_ref, o_ref):
    o_ref[:] = jax.lax.dot_general(
        h_ref[:], w_ref[:], (((1,), (1,)), ((), ())),
        preferred_element_type=jnp.float32) + b_ref[:]


def _edge_conv(x, W, b, k):
    n, d = x.shape
    h = W.shape[0]
    br = _BR
    sq = pl.pallas_call(
        _sq_kernel,
        grid=(n // br,),
        in_specs=[pl.BlockSpec((br, d), lambda p: (p, 0))],
        out_specs=pl.BlockSpec((br, 1), lambda p: (p, 0)),
        out_shape=jax.ShapeDtypeStruct((n, 1), jnp.float32),
    )(x)
    sqr = sq.reshape(1, n)
    b2d = b.reshape(1, h)
    out = pl.pallas_call(
        functools.partial(_knn_conv_kernel, k=k, br=br, n=n),
        grid=(n // br,),
        in_specs=[
            pl.BlockSpec((br, d), lambda p: (p, 0)),
            pl.BlockSpec((n, d), lambda p: (0, 0)),
            pl.BlockSpec((br, 1), lambda p: (p, 0)),
            pl.BlockSpec((1, n), lambda p: (0, 0)),
            pl.BlockSpec((h, d), lambda p: (0, 0)),
            pl.BlockSpec((1, h), lambda p: (0, 0)),
        ],
        out_specs=pl.BlockSpec((br, h), lambda p: (p, 0)),
        out_shape=jax.ShapeDtypeStruct((n, h), jnp.float32),
        scratch_shapes=[
            pltpu.VMEM((br * k, d), jnp.float32),
            pltpu.VMEM((br, k + 1), jnp.int32),
            pltpu.SMEM((br, k + 1), jnp.int32),
            pltpu.SemaphoreType.DMA,
        ],
    )(x, x, sq, sqr, W, b2d)
    return out


def kernel(x, W1, b1, W2, b2, W3, b3):
    h1 = _edge_conv(x, W1, b1, 5)
    h2 = _edge_conv(h1, W2, b2, 10)
    n, h = h2.shape
    c = W3.shape[0]
    br = _BR
    out = pl.pallas_call(
        _out_kernel,
        grid=(n // br,),
        in_specs=[
            pl.BlockSpec((br, h), lambda p: (p, 0)),
            pl.BlockSpec((c, h), lambda p: (0, 0)),
            pl.BlockSpec((1, c), lambda p: (0, 0)),
        ],
        out_specs=pl.BlockSpec((br, c), lambda p: (p, 0)),
        out_shape=jax.ShapeDtypeStruct((n, c), jnp.float32),
    )(h2, W3, b3.reshape(1, c))
    return out


# speedup vs baseline: 1.6657x; 1.2136x over previous
import functools

import jax
import jax.numpy as jnp
from jax import lax
from jax.experimental import pallas as pl
from jax.experimental.pallas import tpu as pltpu
from jax.experimental.pallas import tpu_sc as plsc

_BR = 256
_CS = 128
_RS = 64
_M = 4
_CH = 256

_BIG = 3e38


def _sq_kernel(x_ref, sq_ref):
    xb = x_ref[:]
    sq_ref[:] = jnp.sum(xb * xb, axis=1, keepdims=True)


def _knn_kernel(xb_ref, x_ref, sqb_ref, sqr_ref, idx_ref,
                flag_vmem, flag_smem, sem, *, k, br, n):
    p = pl.program_id(0)
    g = jax.lax.dot_general(
        xb_ref[:], x_ref[:], (((1,), (1,)), ((), ())),
        preferred_element_type=jnp.float32)
    scores = (sqb_ref[:] + sqr_ref[:]) - 2.0 * g
    col = jax.lax.broadcasted_iota(jnp.int32, (br, n), 1)
    row = jax.lax.broadcasted_iota(jnp.int32, (br, n), 0)
    scores = scores + jnp.where(col == row + p * br, 1e10, 0.0)

    nc = n // _CS
    m = _M
    lane = jax.lax.broadcasted_iota(jnp.int32, (_RS, _CS), 1)
    v_rows = []
    gi_rows = []
    for r0 in range(0, br, _RS):
        T = [jnp.full((_RS, _CS), _BIG) for _ in range(m)]
        I = [jnp.zeros((_RS, _CS), jnp.int32) for _ in range(m)]
        for c in range(nc):
            v = jax.lax.slice(scores, (r0, c * _CS), (r0 + _RS, (c + 1) * _CS))
            cc = jnp.full((_RS, _CS), c, jnp.int32)
            ltb = v < T[m - 1]
            cv = jnp.where(ltb, v, T[m - 1])
            ci = jnp.where(ltb, cc, I[m - 1])
            lt = [cv < T[j] for j in range(m - 1)]
            T[m - 1] = jnp.where(lt[m - 2], T[m - 2], cv)
            I[m - 1] = jnp.where(lt[m - 2], I[m - 2], ci)
            for j in range(m - 2, 0, -1):
                T[j] = jnp.where(lt[j - 1], T[j - 1],
                                 jnp.where(lt[j], cv, T[j]))
                I[j] = jnp.where(lt[j - 1], I[j - 1],
                                 jnp.where(lt[j], ci, I[j]))
            T[0] = jnp.where(lt[0], cv, T[0])
            I[0] = jnp.where(lt[0], ci, I[0])
        v_rows.append(jnp.concatenate(T, axis=1))
        gi_rows.append(jnp.concatenate(
            [ij * _CS + lane for ij in I], axis=1))
    cand_v = jnp.concatenate(v_rows, axis=0)
    cand_g = jnp.concatenate(gi_rows, axis=0)

    mm = _M * _CS
    vk = None
    for t in range(k):
        ev = jnp.min(cand_v, axis=1, keepdims=True)
        gsel = jnp.min(jnp.where(cand_v == ev, cand_g, n), axis=1,
                       keepdims=True)
        idx_ref[:, t:t + 1] = gsel
        if t == k - 1:
            vk = ev
        cand_v = jnp.where((cand_v == ev) & (cand_g == gsel), _BIG, cand_v)
    rem = jnp.min(cand_v, axis=1, keepdims=True)
    tie = (vk == rem).astype(jnp.int32)
    selm = (cand_v >= 1e30).astype(jnp.int32)
    lanefull = selm[:, 0:_CS]
    for j in range(1, _M):
        lanefull = lanefull * selm[:, j * _CS:(j + 1) * _CS]
    rowbad = jnp.maximum(jnp.max(lanefull, axis=1, keepdims=True), tie)
    allbad = jnp.max(rowbad, axis=0, keepdims=True)
    flag_vmem[:] = jnp.broadcast_to(allbad, (br, 1))

    cp = pltpu.make_async_copy(flag_vmem, flag_smem, sem)
    cp.start()
    cp.wait()

    @pl.when(flag_smem[0, 0] != 0)
    def _fallback():
        sc = scores
        for t in range(k):
            idxv = jnp.argmin(sc, axis=1, keepdims=True).astype(jnp.int32)
            idx_ref[:, t:t + 1] = idxv
            if t < k - 1:
                sc = jnp.where(col == idxv, _BIG, sc)


def _conv_kernel(*refs, k, br):
    xb_ref = refs[0]
    gath = refs[1:1 + k]
    w_ref, b_ref, o_ref = refs[1 + k], refs[2 + k], refs[3 + k]
    xb = xb_ref[:]
    acc = None
    for t in range(k):
        msg = gath[t][:] - xb
        ht = jax.lax.dot_general(
            msg, w_ref[:], (((1,), (1,)), ((), ())),
            preferred_element_type=jnp.float32)
        acc = ht if acc is None else jnp.maximum(acc, ht)
    o_ref[:] = acc + b_ref[:]


def _out_kernel(h_ref, w_ref, b_ref, o_ref):
    o_ref[:] = jax.lax.dot_general(
        h_ref[:], w_ref[:], (((1,), (1,)), ((), ())),
        preferred_element_type=jnp.float32) + b_ref[:]


def _sc_gather(x, idx_flat):
    b, (v, d) = idx_flat.shape[0], x.shape
    info = plsc.get_sparse_core_info()
    nw = info.num_cores * info.num_subcores
    b_per_w = b // nw
    n_chunks = b_per_w // _CH
    mesh = plsc.VectorSubcoreMesh(core_axis_name="c", subcore_axis_name="s")

    @functools.partial(
        pl.kernel, mesh=mesh,
        out_type=jax.ShapeDtypeStruct((b, d), jnp.float32),
        scratch_types=[
            pltpu.VMEM((_CH,), jnp.int32),
            pltpu.VMEM((_CH, d), jnp.float32),
            pltpu.SemaphoreType.DMA,
        ],
    )
    def gk(table_hbm, idx_hbm, out_hbm, idx_v, rows_v, sem):
        wid = lax.axis_index("s") * info.num_cores + lax.axis_index("c")
        base = wid * b_per_w
        for ci in range(n_chunks):
            off = base + ci * _CH
            pltpu.sync_copy(idx_hbm.at[pl.ds(off, _CH)], idx_v)
            pltpu.async_copy(table_hbm.at[idx_v], rows_v, sem).wait()
            pltpu.sync_copy(rows_v, out_hbm.at[pl.ds(off, _CH)])

    return gk(x, idx_flat)


def _edge_conv(x, W, b, k):
    n, d = x.shape
    h = W.shape[0]
    br = _BR
    sq = pl.pallas_call(
        _sq_kernel,
        grid=(n // br,),
        in_specs=[pl.BlockSpec((br, d), lambda p: (p, 0))],
        out_specs=pl.BlockSpec((br, 1), lambda p: (p, 0)),
        out_shape=jax.ShapeDtypeStruct((n, 1), jnp.float32),
    )(x)
    sqr = sq.reshape(1, n)
    idx = pl.pallas_call(
        functools.partial(_knn_kernel, k=k, br=br, n=n),
        grid=(n // br,),
        in_specs=[
            pl.BlockSpec((br, d), lambda p: (p, 0)),
            pl.BlockSpec((n, d), lambda p: (0, 0)),
            pl.BlockSpec((br, 1), lambda p: (p, 0)),
            pl.BlockSpec((1, n), lambda p: (0, 0)),
        ],
        out_specs=pl.BlockSpec((br, k), lambda p: (p, 0)),
        out_shape=jax.ShapeDtypeStruct((n, k), jnp.int32),
        scratch_shapes=[
            pltpu.VMEM((br, 1), jnp.int32),
            pltpu.SMEM((br, 1), jnp.int32),
            pltpu.SemaphoreType.DMA,
        ],
    )(x, x, sq, sqr)
    idx_tm = idx.T.reshape(k * n)
    gath = _sc_gather(x, idx_tm)
    nb = n // br
    in_specs = [pl.BlockSpec((br, d), lambda p: (p, 0))]
    gargs = []
    for t in range(k):
        in_specs.append(
            pl.BlockSpec((br, d), lambda p, t=t: (t * nb + p, 0)))
        gargs.append(gath)
    in_specs.append(pl.BlockSpec((h, d), lambda p: (0, 0)))
    in_specs.append(pl.BlockSpec((1, h), lambda p: (0, 0)))
    out = pl.pallas_call(
        functools.partial(_conv_kernel, k=k, br=br),
        grid=(nb,),
        in_specs=in_specs,
        out_specs=pl.BlockSpec((br, h), lambda p: (p, 0)),
        out_shape=jax.ShapeDtypeStruct((n, h), jnp.float32),
    )(x, *gargs, W, b.reshape(1, h))
    return out


def kernel(x, W1, b1, W2, b2, W3, b3):
    h1 = _edge_conv(x, W1, b1, 5)
    h2 = _edge_conv(h1, W2, b2, 10)
    n, h = h2.shape
    c = W3.shape[0]
    br = _BR
    out = pl.pallas_call(
        _out_kernel,
        grid=(n // br,),
        in_specs=[
            pl.BlockSpec((br, h), lambda p: (p, 0)),
            pl.BlockSpec((c, h), lambda p: (0, 0)),
            pl.BlockSpec((1, c), lambda p: (0, 0)),
        ],
        out_specs=pl.BlockSpec((br, c), lambda p: (p, 0)),
        out_shape=jax.ShapeDtypeStruct((n, c), jnp.float32),
    )(h2, W3, b3.reshape(1, c))
    return out
